# Initial kernel scaffold; baseline (speedup 1.0000x reference)
import functools

import jax
import jax.numpy as jnp
from jax import lax
from jax.experimental import pallas as pl
from jax.experimental.pallas import tpu as pltpu
from jax.experimental.pallas import tpu_sc as plsc

_BATCH = 16384
_SEQ = 20
_B = _BATCH * _SEQ
_D = 32
_NC = 2
_NS = 16
_NW = _NC * _NS
_BPW = _B // _NW
_C = 1024
_NCHUNK = _BPW // _C


def _make_gather():
    mesh = plsc.VectorSubcoreMesh(core_axis_name="c", subcore_axis_name="s")

    @functools.partial(
        pl.kernel,
        mesh=mesh,
        out_type=jax.ShapeDtypeStruct((_B, _D), jnp.float32),
        scratch_types=[
            pltpu.VMEM((_BPW,), jnp.int32),
            pltpu.VMEM((2, _C, _D), jnp.float32),
            pltpu.SemaphoreType.DMA,
            pltpu.SemaphoreType.DMA,
        ],
    )
    def gather_kernel(idx_hbm, table_hbm, out_hbm, idx_v, rows_v, sem0, sem1):
        wid = lax.axis_index("s") * _NC + lax.axis_index("c")
        base = wid * _BPW
        pltpu.sync_copy(idx_hbm.at[pl.ds(base, _BPW)], idx_v)
        sems = (sem0, sem1)

        def start(i, b):
            return pltpu.async_copy(
                table_hbm.at[idx_v.at[pl.ds(i * _C, _C)]],
                rows_v.at[b],
                sems[b],
            )

        inflight = [start(0, 0), None]
        for i in range(_NCHUNK):
            b = i % 2
            inflight[b].wait()
            if i + 1 < _NCHUNK:
                inflight[1 - b] = start(i + 1, 1 - b)
            pltpu.sync_copy(rows_v.at[b], out_hbm.at[pl.ds(base + i * _C, _C)])

    return gather_kernel


_gather = _make_gather()


def kernel(x, word_embedding):
    idx = x.reshape(_B).astype(jnp.int32)
    out = _gather(idx, word_embedding)
    return out.reshape(x.shape[0], x.shape[1], _D)


# speedup vs baseline: 1.5054x; 1.5054x over previous
import functools

import jax
import jax.numpy as jnp
from jax import lax
from jax.experimental import pallas as pl
from jax.experimental.pallas import tpu as pltpu
from jax.experimental.pallas import tpu_sc as plsc

_BATCH = 16384
_SEQ = 20
_B = _BATCH * _SEQ
_D = 32
_NC = 2
_NS = 16
_NW = _NC * _NS
_BPW = _B // _NW
_C = 1024
_NCHUNK = _BPW // _C


def _make_gather():
    mesh = plsc.VectorSubcoreMesh(core_axis_name="c", subcore_axis_name="s")

    @functools.partial(
        pl.kernel,
        mesh=mesh,
        compiler_params=pltpu.CompilerParams(use_tc_tiling_on_sc=False),
        out_type=jax.ShapeDtypeStruct((_B, _D), jnp.float32),
        scratch_types=[
            pltpu.VMEM((_BPW,), jnp.int32),
            pltpu.VMEM((2, _C, _D), jnp.float32),
            pltpu.SemaphoreType.DMA,
            pltpu.SemaphoreType.DMA,
        ],
    )
    def gather_kernel(idx_hbm, table_hbm, out_hbm, idx_v, rows_v, sem0, sem1):
        wid = lax.axis_index("s") * _NC + lax.axis_index("c")
        base = wid * _BPW
        pltpu.sync_copy(idx_hbm.at[pl.ds(base, _BPW)], idx_v)
        sems = (sem0, sem1)

        def start(i, b):
            return pltpu.async_copy(
                table_hbm.at[idx_v.at[pl.ds(i * _C, _C)]],
                rows_v.at[b],
                sems[b],
            )

        inflight = [start(0, 0), None]
        for i in range(_NCHUNK):
            b = i % 2
            inflight[b].wait()
            if i + 1 < _NCHUNK:
                inflight[1 - b] = start(i + 1, 1 - b)
            pltpu.sync_copy(rows_v.at[b], out_hbm.at[pl.ds(base + i * _C, _C)])

    return gather_kernel


_gather = _make_gather()


def kernel(x, word_embedding):
    idx = x.reshape(_B).astype(jnp.int32)
    out = _gather(idx, word_embedding)
    return out.reshape(x.shape[0], x.shape[1], _D)
